# baseline (speedup 1.0000x reference)
import jax
import jax.numpy as jnp
from jax import lax
from jax.experimental import pallas as pl
from jax.experimental.pallas import tpu as pltpu

_CONF = 0.25
_IOU = 0.45
_MAXWH = 4096.0
_MAXDET = 300
_B, _N, _C = 4, 5000, 80
_T = 128
_NBLK = 40
_NP = _T * _NBLK
_G = 512
_NGRP = _NP // _G


def _score_body(pred_ref, out_ref):
    p = pred_ref[...]
    obj = p[:, 4:5]
    cls = p[:, 5:] * obj
    conf = jnp.max(cls, axis=1, keepdims=True)
    j = jnp.argmax(cls, axis=1).astype(jnp.float32)[:, None]
    xy = p[:, 0:2]
    half = p[:, 2:4] * 0.5
    b1 = xy - half
    b2 = xy + half
    valid = (obj > _CONF) & (conf > _CONF)
    score = jnp.where(valid, conf, -1.0)
    off = j * _MAXWH
    bo1 = jnp.where(valid, b1 + off, 0.0)
    bo2 = jnp.where(valid, b2 + off, 0.0)
    zero = jnp.zeros_like(conf)
    out_ref[...] = jnp.concatenate(
        [bo1, bo2, score, b1, b2, conf, j, zero, zero, zero, zero, zero],
        axis=1,
    )


def _colmat(v):
    ones = jnp.ones((1, 1), jnp.float32)
    return lax.dot_general(
        v, ones, (((0,), (0,)), ((), ())),
        precision=lax.Precision.HIGHEST,
        preferred_element_type=jnp.float32,
    )


def _iou_gt(px1, py1, px2, py2, parea, tx1, ty1, tx2, ty2, tarea):
    ltx = jnp.maximum(px1, tx1)
    lty = jnp.maximum(py1, ty1)
    rbx = jnp.minimum(px2, tx2)
    rby = jnp.minimum(py2, ty2)
    iw = jnp.maximum(rbx - ltx, 0.0)
    ih = jnp.maximum(rby - lty, 0.0)
    inter = iw * ih
    union = parea + tarea - inter
    return (inter / (union + 1e-9)) > _IOU


def _nms_body(c_ref, c2_ref, keep_ref, sup_ref, area2_ref):
    sup_ref[...] = jnp.zeros((_NBLK, 1, _T), jnp.float32)
    keep_ref[...] = jnp.zeros((_NBLK, 1, _T), jnp.float32)
    area2_ref[...] = (jnp.maximum(c2_ref[2] - c2_ref[0], 0.0)
                      * jnp.maximum(c2_ref[3] - c2_ref[1], 0.0))
    starts = c_ref[4][:, :, 0:1]
    nvb = jnp.sum(jnp.where(starts > 0.0, 1, 0))
    g_hi = (nvb + 3) // 4

    def block_body(k, _):
        tx1 = c_ref[0, k]
        ty1 = c_ref[1, k]
        tx2 = c_ref[2, k]
        ty2 = c_ref[3, k]
        tsc = c_ref[4, k]
        tarea = jnp.maximum(tx2 - tx1, 0.0) * jnp.maximum(ty2 - ty1, 0.0)
        px1 = _colmat(tx1)
        py1 = _colmat(ty1)
        px2 = _colmat(tx2)
        py2 = _colmat(ty2)
        parea = _colmat(tarea)
        s_gt = _iou_gt(px1, py1, px2, py2, parea, tx1, ty1, tx2, ty2, tarea)
        rows = lax.broadcasted_iota(jnp.int32, (_T, _T), 0)
        cols = lax.broadcasted_iota(jnp.int32, (_T, _T), 1)
        s_mat = jnp.where(s_gt & (rows < cols), 1.0, 0.0)
        active = jnp.where((tsc > 0.0) & (sup_ref[k] < 0.5), 1.0, 0.0)

        def fix_body(carry):
            _, cur = carry
            curc = _colmat(cur)
            hitc = jnp.max(s_mat * curc, axis=0, keepdims=True)
            nxt = active * jnp.where(hitc > 0.5, 0.0, 1.0)
            return (cur, nxt)

        def fix_cond(carry):
            prev, cur = carry
            return jnp.any(prev != cur)

        first = fix_body((active, active))
        _, keepk = lax.while_loop(fix_cond, fix_body, first)
        keep_ref[k] = keepk
        keepc = _colmat(keepk)

        g0 = (k + 1) * _T // _G

        def cross_body(g, _):
            ux1 = c2_ref[0, g]
            uy1 = c2_ref[1, g]
            ux2 = c2_ref[2, g]
            uy2 = c2_ref[3, g]
            uarea = area2_ref[g]
            hit = _iou_gt(px1, py1, px2, py2, parea, ux1, uy1, ux2, uy2, uarea)
            supg = jnp.max(jnp.where(hit, 1.0, 0.0) * keepc, axis=0, keepdims=True)
            for i in range(_G // _T):
                row = g * (_G // _T) + i
                sup_ref[row] = jnp.maximum(sup_ref[row], supg[:, i * _T:(i + 1) * _T])
            return 0

        lax.fori_loop(g0, g_hi, cross_body, 0)
        return 0

    lax.fori_loop(0, nvb, block_body, 0)


def kernel(prediction, logits):
    B, N = _B, _N
    pred2 = prediction.reshape(B * N, 85)
    packed = pl.pallas_call(
        _score_body,
        grid=(10,),
        in_specs=[pl.BlockSpec((B * N // 10, 85), lambda i: (i, 0))],
        out_specs=pl.BlockSpec((B * N // 10, 16), lambda i: (i, 0)),
        out_shape=jax.ShapeDtypeStruct((B * N, 16), jnp.float32),
    )(pred2)
    pk3 = packed.reshape(B, N, 16)
    scores = pk3[:, :, 4]
    order = jnp.argsort(-scores, axis=1)
    srt = jnp.take_along_axis(pk3[:, :, 0:5], order[:, :, None], axis=1)
    pad = jnp.concatenate(
        [jnp.zeros((B, _NP - N, 4), jnp.float32),
         jnp.full((B, _NP - N, 1), -1.0, jnp.float32)], axis=2)
    srt = jnp.concatenate([srt, pad], axis=1)
    carr = jnp.moveaxis(srt, 2, 1)
    c1 = carr.reshape(B, 5, _NBLK, 1, _T)
    c2 = carr.reshape(B, 5, _NGRP, 1, _G)

    keep_f = pl.pallas_call(
        _nms_body,
        grid=(B,),
        in_specs=[
            pl.BlockSpec((None, 5, _NBLK, 1, _T), lambda b: (b, 0, 0, 0, 0)),
            pl.BlockSpec((None, 5, _NGRP, 1, _G), lambda b: (b, 0, 0, 0, 0)),
        ],
        out_specs=pl.BlockSpec((None, _NBLK, 1, _T), lambda b: (b, 0, 0, 0)),
        out_shape=jax.ShapeDtypeStruct((B, _NBLK, 1, _T), jnp.float32),
        scratch_shapes=[pltpu.VMEM((_NBLK, 1, _T), jnp.float32),
                        pltpu.VMEM((_NGRP, 1, _G), jnp.float32)],
    )(c1, c2)

    keep = keep_f.reshape(B, _NP)[:, :N] > 0.5
    pos = jnp.arange(N, dtype=jnp.int32)
    ck = jnp.cumsum(keep.astype(jnp.int32), axis=1)
    ktot = ck[:, -1:]
    rank = jnp.where(keep, ck - 1, ktot + pos[None, :] - ck)
    bidx = jnp.arange(B, dtype=jnp.int32)[:, None]
    sel_slot = jnp.zeros((B, _MAXDET), jnp.int32).at[
        bidx, rank].set(jnp.broadcast_to(pos, (B, N)), mode="drop")
    vals = jnp.take_along_axis(keep, sel_slot, axis=1)
    sel = jnp.take_along_axis(order, sel_slot, axis=1)
    det_base = pk3[:, :, 5:11]
    dets = jnp.take_along_axis(det_base, sel[:, :, None], axis=1)
    logs = jnp.take_along_axis(logits, sel[:, :, None], axis=1)
    return (dets, logs, vals)


# speedup vs baseline: 1.0738x; 1.0738x over previous
import functools

import jax
import jax.numpy as jnp
from jax import lax
from jax.experimental import pallas as pl
from jax.experimental.pallas import tpu as pltpu
from jax.experimental.pallas import tpu_sc as plsc

_CONF = 0.25
_IOU = 0.45
_MAXWH = 4096.0
_MAXDET = 300
_B, _N, _C = 4, 5000, 80
_T = 128
_NBLK = 40
_NP = _T * _NBLK
_G = 512
_NGRP = _NP // _G


def _score_body(pred_ref, out_ref):
    p = pred_ref[...]
    obj = p[:, 4:5]
    cls = p[:, 5:] * obj
    conf = jnp.max(cls, axis=1, keepdims=True)
    j = jnp.argmax(cls, axis=1).astype(jnp.float32)[:, None]
    xy = p[:, 0:2]
    half = p[:, 2:4] * 0.5
    b1 = xy - half
    b2 = xy + half
    valid = (obj > _CONF) & (conf > _CONF)
    score = jnp.where(valid, conf, -1.0)
    off = j * _MAXWH
    bo1 = jnp.where(valid, b1 + off, 0.0)
    bo2 = jnp.where(valid, b2 + off, 0.0)
    zero = jnp.zeros_like(conf)
    out_ref[...] = jnp.concatenate(
        [bo1, bo2, score, b1, b2, conf, j, zero, zero, zero, zero, zero],
        axis=1,
    )


def _colmat(v):
    ones = jnp.ones((1, 1), jnp.float32)
    return lax.dot_general(
        v, ones, (((0,), (0,)), ((), ())),
        precision=lax.Precision.HIGHEST,
        preferred_element_type=jnp.float32,
    )


def _iou_gt(px1, py1, px2, py2, parea, tx1, ty1, tx2, ty2, tarea):
    ltx = jnp.maximum(px1, tx1)
    lty = jnp.maximum(py1, ty1)
    rbx = jnp.minimum(px2, tx2)
    rby = jnp.minimum(py2, ty2)
    iw = jnp.maximum(rbx - ltx, 0.0)
    ih = jnp.maximum(rby - lty, 0.0)
    inter = iw * ih
    union = parea + tarea - inter
    return (inter / (union + 1e-9)) > _IOU


def _nms_body(c_ref, c2_ref, keep_ref, sup_ref, area2_ref):
    sup_ref[...] = jnp.zeros((_NBLK, 1, _T), jnp.float32)
    keep_ref[...] = jnp.zeros((_NBLK, 1, _T), jnp.float32)
    area2_ref[...] = (jnp.maximum(c2_ref[2] - c2_ref[0], 0.0)
                      * jnp.maximum(c2_ref[3] - c2_ref[1], 0.0))
    starts = c_ref[4][:, :, 0:1]
    nvb = jnp.sum(jnp.where(starts > 0.0, 1, 0))
    g_hi = (nvb + 3) // 4

    def block_body(k, _):
        tx1 = c_ref[0, k]
        ty1 = c_ref[1, k]
        tx2 = c_ref[2, k]
        ty2 = c_ref[3, k]
        tsc = c_ref[4, k]
        tarea = jnp.maximum(tx2 - tx1, 0.0) * jnp.maximum(ty2 - ty1, 0.0)
        px1 = _colmat(tx1)
        py1 = _colmat(ty1)
        px2 = _colmat(tx2)
        py2 = _colmat(ty2)
        parea = _colmat(tarea)
        s_gt = _iou_gt(px1, py1, px2, py2, parea, tx1, ty1, tx2, ty2, tarea)
        rows = lax.broadcasted_iota(jnp.int32, (_T, _T), 0)
        cols = lax.broadcasted_iota(jnp.int32, (_T, _T), 1)
        s_mat = jnp.where(s_gt & (rows < cols), 1.0, 0.0)
        active = jnp.where((tsc > 0.0) & (sup_ref[k] < 0.5), 1.0, 0.0)

        def fix_body(carry):
            _, cur = carry
            curc = _colmat(cur)
            hitc = jnp.max(s_mat * curc, axis=0, keepdims=True)
            nxt = active * jnp.where(hitc > 0.5, 0.0, 1.0)
            return (cur, nxt)

        def fix_cond(carry):
            prev, cur = carry
            return jnp.any(prev != cur)

        first = fix_body((active, active))
        _, keepk = lax.while_loop(fix_cond, fix_body, first)
        keep_ref[k] = keepk
        keepc = _colmat(keepk)

        g0 = (k + 1) * _T // _G

        def cross_body(g, _):
            ux1 = c2_ref[0, g]
            uy1 = c2_ref[1, g]
            ux2 = c2_ref[2, g]
            uy2 = c2_ref[3, g]
            uarea = area2_ref[g]
            hit = _iou_gt(px1, py1, px2, py2, parea, ux1, uy1, ux2, uy2, uarea)
            supg = jnp.max(jnp.where(hit, 1.0, 0.0) * keepc, axis=0, keepdims=True)
            for i in range(_G // _T):
                row = g * (_G // _T) + i
                sup_ref[row] = jnp.maximum(sup_ref[row], supg[:, i * _T:(i + 1) * _T])
            return 0

        lax.fori_loop(g0, g_hi, cross_body, 0)
        return 0

    lax.fori_loop(0, nvb, block_body, 0)


_NSEL = 320
_NW = 32
_RPW = _B * _NSEL // _NW


@functools.partial(
    pl.kernel,
    mesh=plsc.VectorSubcoreMesh(core_axis_name="c", subcore_axis_name="s"),
    out_type=jax.ShapeDtypeStruct((_B * _NSEL, 128), jnp.float32),
    scratch_types=[
        pltpu.VMEM((_RPW,), jnp.int32),
        pltpu.VMEM((_RPW, 128), jnp.float32),
        pltpu.SemaphoreType.DMA,
    ],
)
def _sc_gather(sel_hbm, tab_hbm, out_hbm, idx_v, rows_v, sem):
    wid = lax.axis_index("s") * 2 + lax.axis_index("c")
    base = wid * _RPW
    pltpu.sync_copy(sel_hbm.at[pl.ds(base, _RPW)], idx_v)
    pltpu.async_copy(tab_hbm.at[idx_v], rows_v, sem).wait()
    pltpu.sync_copy(rows_v, out_hbm.at[pl.ds(base, _RPW)])


def kernel(prediction, logits):
    B, N = _B, _N
    pred2 = prediction.reshape(B * N, 85)
    packed = pl.pallas_call(
        _score_body,
        grid=(10,),
        in_specs=[pl.BlockSpec((B * N // 10, 85), lambda i: (i, 0))],
        out_specs=pl.BlockSpec((B * N // 10, 16), lambda i: (i, 0)),
        out_shape=jax.ShapeDtypeStruct((B * N, 16), jnp.float32),
    )(pred2)
    pk3 = packed.reshape(B, N, 16)
    scores = pk3[:, :, 4]
    order = jnp.argsort(-scores, axis=1)
    srt = jnp.take_along_axis(pk3[:, :, 0:5], order[:, :, None], axis=1)
    pad = jnp.concatenate(
        [jnp.zeros((B, _NP - N, 4), jnp.float32),
         jnp.full((B, _NP - N, 1), -1.0, jnp.float32)], axis=2)
    srt = jnp.concatenate([srt, pad], axis=1)
    carr = jnp.moveaxis(srt, 2, 1)
    c1 = carr.reshape(B, 5, _NBLK, 1, _T)
    c2 = carr.reshape(B, 5, _NGRP, 1, _G)

    keep_f = pl.pallas_call(
        _nms_body,
        grid=(B,),
        in_specs=[
            pl.BlockSpec((None, 5, _NBLK, 1, _T), lambda b: (b, 0, 0, 0, 0)),
            pl.BlockSpec((None, 5, _NGRP, 1, _G), lambda b: (b, 0, 0, 0, 0)),
        ],
        out_specs=pl.BlockSpec((None, _NBLK, 1, _T), lambda b: (b, 0, 0, 0)),
        out_shape=jax.ShapeDtypeStruct((B, _NBLK, 1, _T), jnp.float32),
        scratch_shapes=[pltpu.VMEM((_NBLK, 1, _T), jnp.float32),
                        pltpu.VMEM((_NGRP, 1, _G), jnp.float32)],
    )(c1, c2)

    keep = keep_f.reshape(B, _NP)[:, :N] > 0.5
    pos = jnp.arange(N, dtype=jnp.int32)
    ck = jnp.cumsum(keep.astype(jnp.int32), axis=1)
    ktot = ck[:, -1:]
    rank = jnp.where(keep, ck - 1, ktot + pos[None, :] - ck)
    bidx = jnp.arange(B, dtype=jnp.int32)[:, None]
    sel = jnp.zeros((B, _MAXDET), jnp.int32).at[
        bidx, rank].set(order.astype(jnp.int32), mode="drop")
    vals = jnp.arange(_MAXDET, dtype=jnp.int32)[None, :] < ktot
    flat = sel + (jnp.arange(B, dtype=jnp.int32) * N)[:, None]
    flat_pad = jnp.concatenate(
        [flat, jnp.zeros((B, _NSEL - _MAXDET), jnp.int32)], axis=1).reshape(-1)
    table = jnp.concatenate(
        [logits.reshape(B * N, _C), packed,
         jnp.zeros((B * N, 128 - _C - 16), jnp.float32)], axis=1)
    rows = _sc_gather(flat_pad, table)
    rows = rows.reshape(B, _NSEL, 128)
    logs = rows[:, :_MAXDET, :_C]
    dets = rows[:, :_MAXDET, _C + 5:_C + 11]
    return (dets, logs, vals)


# speedup vs baseline: 1.1659x; 1.0858x over previous
import functools

import jax
import jax.numpy as jnp
from jax import lax
from jax.experimental import pallas as pl
from jax.experimental.pallas import tpu as pltpu
from jax.experimental.pallas import tpu_sc as plsc

_CONF = 0.25
_IOU = 0.45
_MAXWH = 4096.0
_MAXDET = 300
_B, _N, _C = 4, 5000, 80
_T = 256
_NBLK = 20
_NP = _T * _NBLK
_G = 512
_NGRP = _NP // _G


def _score_body(pred_ref, out_ref):
    p = pred_ref[...]
    obj = p[:, 4:5]
    cls = p[:, 5:] * obj
    conf = jnp.max(cls, axis=1, keepdims=True)
    j = jnp.argmax(cls, axis=1).astype(jnp.float32)[:, None]
    xy = p[:, 0:2]
    half = p[:, 2:4] * 0.5
    b1 = xy - half
    b2 = xy + half
    valid = (obj > _CONF) & (conf > _CONF)
    score = jnp.where(valid, conf, -1.0)
    off = j * _MAXWH
    bo1 = jnp.where(valid, b1 + off, 0.0)
    bo2 = jnp.where(valid, b2 + off, 0.0)
    zero = jnp.zeros_like(conf)
    out_ref[...] = jnp.concatenate(
        [bo1, bo2, score, b1, b2, conf, j, zero, zero, zero, zero, zero],
        axis=1,
    )


def _colmat(v):
    ones = jnp.ones((1, 1), jnp.float32)
    return lax.dot_general(
        v, ones, (((0,), (0,)), ((), ())),
        precision=lax.Precision.HIGHEST,
        preferred_element_type=jnp.float32,
    )


def _iou_gt(px1, py1, px2, py2, parea, tx1, ty1, tx2, ty2, tarea):
    ltx = jnp.maximum(px1, tx1)
    lty = jnp.maximum(py1, ty1)
    rbx = jnp.minimum(px2, tx2)
    rby = jnp.minimum(py2, ty2)
    iw = jnp.maximum(rbx - ltx, 0.0)
    ih = jnp.maximum(rby - lty, 0.0)
    inter = iw * ih
    union = parea + tarea - inter
    return (inter / (union + 1e-9)) > _IOU


def _nms_body(c_ref, c2_ref, keep_ref, sup_ref, area2_ref):
    sup_ref[...] = jnp.zeros((_NBLK, 1, _T), jnp.float32)
    keep_ref[...] = jnp.zeros((_NBLK, 1, _T), jnp.float32)
    area2_ref[...] = (jnp.maximum(c2_ref[2] - c2_ref[0], 0.0)
                      * jnp.maximum(c2_ref[3] - c2_ref[1], 0.0))
    starts = c_ref[4][:, :, 0:1]
    nvb = jnp.sum(jnp.where(starts > 0.0, 1, 0))
    g_hi = (nvb * _T + _G - 1) // _G

    def block_body(k, _):
        tx1 = c_ref[0, k]
        ty1 = c_ref[1, k]
        tx2 = c_ref[2, k]
        ty2 = c_ref[3, k]
        tsc = c_ref[4, k]
        tarea = jnp.maximum(tx2 - tx1, 0.0) * jnp.maximum(ty2 - ty1, 0.0)
        px1 = _colmat(tx1)
        py1 = _colmat(ty1)
        px2 = _colmat(tx2)
        py2 = _colmat(ty2)
        parea = _colmat(tarea)
        s_gt = _iou_gt(px1, py1, px2, py2, parea, tx1, ty1, tx2, ty2, tarea)
        rows = lax.broadcasted_iota(jnp.int32, (_T, _T), 0)
        cols = lax.broadcasted_iota(jnp.int32, (_T, _T), 1)
        s_mat = jnp.where(s_gt & (rows < cols), 1.0, 0.0)
        active = jnp.where((tsc > 0.0) & (sup_ref[k] < 0.5), 1.0, 0.0)

        def fix_body(carry):
            _, cur = carry
            curc = _colmat(cur)
            hitc = jnp.max(s_mat * curc, axis=0, keepdims=True)
            nxt = active * jnp.where(hitc > 0.5, 0.0, 1.0)
            return (cur, nxt)

        def fix_cond(carry):
            prev, cur = carry
            return jnp.any(prev != cur)

        first = fix_body((active, active))
        _, keepk = lax.while_loop(fix_cond, fix_body, first)
        keep_ref[k] = keepk
        keepc = _colmat(keepk)

        g0 = (k + 1) * _T // _G

        def cross_body(g, _):
            ux1 = c2_ref[0, g]
            uy1 = c2_ref[1, g]
            ux2 = c2_ref[2, g]
            uy2 = c2_ref[3, g]
            uarea = area2_ref[g]
            hit = _iou_gt(px1, py1, px2, py2, parea, ux1, uy1, ux2, uy2, uarea)
            supg = jnp.max(jnp.where(hit, 1.0, 0.0) * keepc, axis=0, keepdims=True)
            for i in range(_G // _T):
                row = g * (_G // _T) + i
                sup_ref[row] = jnp.maximum(sup_ref[row], supg[:, i * _T:(i + 1) * _T])
            return 0

        lax.fori_loop(g0, g_hi, cross_body, 0)
        return 0

    lax.fori_loop(0, nvb, block_body, 0)


_NSEL = 320
_NW = 32
_RPW = _B * _NSEL // _NW


@functools.partial(
    pl.kernel,
    mesh=plsc.VectorSubcoreMesh(core_axis_name="c", subcore_axis_name="s"),
    out_type=jax.ShapeDtypeStruct((_B * _NSEL, 128), jnp.float32),
    scratch_types=[
        pltpu.VMEM((_RPW,), jnp.int32),
        pltpu.VMEM((_RPW, 128), jnp.float32),
        pltpu.SemaphoreType.DMA,
    ],
)
def _sc_gather(sel_hbm, tab_hbm, out_hbm, idx_v, rows_v, sem):
    wid = lax.axis_index("s") * 2 + lax.axis_index("c")
    base = wid * _RPW
    pltpu.sync_copy(sel_hbm.at[pl.ds(base, _RPW)], idx_v)
    pltpu.async_copy(tab_hbm.at[idx_v], rows_v, sem).wait()
    pltpu.sync_copy(rows_v, out_hbm.at[pl.ds(base, _RPW)])


def kernel(prediction, logits):
    B, N = _B, _N
    pred2 = prediction.reshape(B * N, 85)
    packed = pl.pallas_call(
        _score_body,
        grid=(10,),
        in_specs=[pl.BlockSpec((B * N // 10, 85), lambda i: (i, 0))],
        out_specs=pl.BlockSpec((B * N // 10, 16), lambda i: (i, 0)),
        out_shape=jax.ShapeDtypeStruct((B * N, 16), jnp.float32),
    )(pred2)
    pk3 = packed.reshape(B, N, 16)
    scores = pk3[:, :, 4]
    order = jnp.argsort(-scores, axis=1)
    srt = jnp.take_along_axis(pk3[:, :, 0:5], order[:, :, None], axis=1)
    pad = jnp.concatenate(
        [jnp.zeros((B, _NP - N, 4), jnp.float32),
         jnp.full((B, _NP - N, 1), -1.0, jnp.float32)], axis=2)
    srt = jnp.concatenate([srt, pad], axis=1)
    carr = jnp.moveaxis(srt, 2, 1)
    c1 = carr.reshape(B, 5, _NBLK, 1, _T)
    c2 = carr.reshape(B, 5, _NGRP, 1, _G)

    keep_f = pl.pallas_call(
        _nms_body,
        grid=(B,),
        in_specs=[
            pl.BlockSpec((None, 5, _NBLK, 1, _T), lambda b: (b, 0, 0, 0, 0)),
            pl.BlockSpec((None, 5, _NGRP, 1, _G), lambda b: (b, 0, 0, 0, 0)),
        ],
        out_specs=pl.BlockSpec((None, _NBLK, 1, _T), lambda b: (b, 0, 0, 0)),
        out_shape=jax.ShapeDtypeStruct((B, _NBLK, 1, _T), jnp.float32),
        scratch_shapes=[pltpu.VMEM((_NBLK, 1, _T), jnp.float32),
                        pltpu.VMEM((_NGRP, 1, _G), jnp.float32)],
    )(c1, c2)

    keep = keep_f.reshape(B, _NP)[:, :N] > 0.5
    pos = jnp.arange(N, dtype=jnp.int32)
    ck = jnp.cumsum(keep.astype(jnp.int32), axis=1)
    ktot = ck[:, -1:]
    rank = jnp.where(keep, ck - 1, ktot + pos[None, :] - ck)
    bidx = jnp.arange(B, dtype=jnp.int32)[:, None]
    sel = jnp.zeros((B, _MAXDET), jnp.int32).at[
        bidx, rank].set(order.astype(jnp.int32), mode="drop")
    vals = jnp.arange(_MAXDET, dtype=jnp.int32)[None, :] < ktot
    flat = sel + (jnp.arange(B, dtype=jnp.int32) * N)[:, None]
    flat_pad = jnp.concatenate(
        [flat, jnp.zeros((B, _NSEL - _MAXDET), jnp.int32)], axis=1).reshape(-1)
    table = jnp.concatenate(
        [logits.reshape(B * N, _C), packed,
         jnp.zeros((B * N, 128 - _C - 16), jnp.float32)], axis=1)
    rows = _sc_gather(flat_pad, table)
    rows = rows.reshape(B, _NSEL, 128)
    logs = rows[:, :_MAXDET, :_C]
    dets = rows[:, :_MAXDET, _C + 5:_C + 11]
    return (dets, logs, vals)


# speedup vs baseline: 1.2230x; 1.0490x over previous
import functools

import jax
import jax.numpy as jnp
from jax import lax
from jax.experimental import pallas as pl
from jax.experimental.pallas import tpu as pltpu
from jax.experimental.pallas import tpu_sc as plsc

_CONF = 0.25
_IOU = 0.45
_MAXWH = 4096.0
_MAXDET = 300
_B, _N, _C = 4, 5000, 80
_T = 512
_NBLK = 10
_NP = _T * _NBLK
_G = 512
_NGRP = _NP // _G


def _score_body(pred_ref, out_ref):
    p = pred_ref[...]
    obj = p[:, 4:5]
    cls = p[:, 5:] * obj
    conf = jnp.max(cls, axis=1, keepdims=True)
    j = jnp.argmax(cls, axis=1).astype(jnp.float32)[:, None]
    xy = p[:, 0:2]
    half = p[:, 2:4] * 0.5
    b1 = xy - half
    b2 = xy + half
    valid = (obj > _CONF) & (conf > _CONF)
    score = jnp.where(valid, conf, -1.0)
    off = j * _MAXWH
    bo1 = jnp.where(valid, b1 + off, 0.0)
    bo2 = jnp.where(valid, b2 + off, 0.0)
    zero = jnp.zeros_like(conf)
    out_ref[...] = jnp.concatenate(
        [bo1, bo2, score, b1, b2, conf, j, zero, zero, zero, zero, zero],
        axis=1,
    )


def _colmat(v):
    ones = jnp.ones((1, 1), jnp.float32)
    return lax.dot_general(
        v, ones, (((0,), (0,)), ((), ())),
        precision=lax.Precision.HIGHEST,
        preferred_element_type=jnp.float32,
    )


def _iou_gt(px1, py1, px2, py2, parea, tx1, ty1, tx2, ty2, tarea):
    ltx = jnp.maximum(px1, tx1)
    lty = jnp.maximum(py1, ty1)
    rbx = jnp.minimum(px2, tx2)
    rby = jnp.minimum(py2, ty2)
    iw = jnp.maximum(rbx - ltx, 0.0)
    ih = jnp.maximum(rby - lty, 0.0)
    inter = iw * ih
    union = parea + tarea - inter
    return (inter / (union + 1e-9)) > _IOU


def _nms_body(c_ref, c2_ref, keep_ref, sup_ref, area2_ref):
    sup_ref[...] = jnp.zeros((_NBLK, 1, _T), jnp.float32)
    keep_ref[...] = jnp.zeros((_NBLK, 1, _T), jnp.float32)
    area2_ref[...] = (jnp.maximum(c2_ref[2] - c2_ref[0], 0.0)
                      * jnp.maximum(c2_ref[3] - c2_ref[1], 0.0))
    starts = c_ref[4][:, :, 0:1]
    nvb = jnp.sum(jnp.where(starts > 0.0, 1, 0))
    g_hi = (nvb * _T + _G - 1) // _G

    def block_body(k, _):
        tx1 = c_ref[0, k]
        ty1 = c_ref[1, k]
        tx2 = c_ref[2, k]
        ty2 = c_ref[3, k]
        tsc = c_ref[4, k]
        tarea = jnp.maximum(tx2 - tx1, 0.0) * jnp.maximum(ty2 - ty1, 0.0)
        px1 = _colmat(tx1)
        py1 = _colmat(ty1)
        px2 = _colmat(tx2)
        py2 = _colmat(ty2)
        parea = _colmat(tarea)
        s_gt = _iou_gt(px1, py1, px2, py2, parea, tx1, ty1, tx2, ty2, tarea)
        rows = lax.broadcasted_iota(jnp.int32, (_T, _T), 0)
        cols = lax.broadcasted_iota(jnp.int32, (_T, _T), 1)
        s_mat = jnp.where(s_gt & (rows < cols), 1.0, 0.0)
        active = jnp.where((tsc > 0.0) & (sup_ref[k] < 0.5), 1.0, 0.0)

        def fix_body(carry):
            _, cur = carry
            curc = _colmat(cur)
            hitc = jnp.max(s_mat * curc, axis=0, keepdims=True)
            nxt = active * jnp.where(hitc > 0.5, 0.0, 1.0)
            return (cur, nxt)

        def fix_cond(carry):
            prev, cur = carry
            return jnp.any(prev != cur)

        first = fix_body((active, active))
        _, keepk = lax.while_loop(fix_cond, fix_body, first)
        keep_ref[k] = keepk
        keepc = _colmat(keepk)

        g0 = (k + 1) * _T // _G

        def cross_body(g, _):
            ux1 = c2_ref[0, g]
            uy1 = c2_ref[1, g]
            ux2 = c2_ref[2, g]
            uy2 = c2_ref[3, g]
            uarea = area2_ref[g]
            hit = _iou_gt(px1, py1, px2, py2, parea, ux1, uy1, ux2, uy2, uarea)
            supg = jnp.max(jnp.where(hit, 1.0, 0.0) * keepc, axis=0, keepdims=True)
            for i in range(_G // _T):
                row = g * (_G // _T) + i
                sup_ref[row] = jnp.maximum(sup_ref[row], supg[:, i * _T:(i + 1) * _T])
            return 0

        lax.fori_loop(g0, g_hi, cross_body, 0)
        return 0

    lax.fori_loop(0, nvb, block_body, 0)


_NSEL = 320
_NW = 32
_RPW = _B * _NSEL // _NW


@functools.partial(
    pl.kernel,
    mesh=plsc.VectorSubcoreMesh(core_axis_name="c", subcore_axis_name="s"),
    out_type=jax.ShapeDtypeStruct((_B * _NSEL, 128), jnp.float32),
    scratch_types=[
        pltpu.VMEM((_RPW,), jnp.int32),
        pltpu.VMEM((_RPW, 128), jnp.float32),
        pltpu.SemaphoreType.DMA,
    ],
)
def _sc_gather(sel_hbm, tab_hbm, out_hbm, idx_v, rows_v, sem):
    wid = lax.axis_index("s") * 2 + lax.axis_index("c")
    base = wid * _RPW
    pltpu.sync_copy(sel_hbm.at[pl.ds(base, _RPW)], idx_v)
    pltpu.async_copy(tab_hbm.at[idx_v], rows_v, sem).wait()
    pltpu.sync_copy(rows_v, out_hbm.at[pl.ds(base, _RPW)])


def kernel(prediction, logits):
    B, N = _B, _N
    pred2 = prediction.reshape(B * N, 85)
    packed = pl.pallas_call(
        _score_body,
        grid=(10,),
        in_specs=[pl.BlockSpec((B * N // 10, 85), lambda i: (i, 0))],
        out_specs=pl.BlockSpec((B * N // 10, 16), lambda i: (i, 0)),
        out_shape=jax.ShapeDtypeStruct((B * N, 16), jnp.float32),
    )(pred2)
    pk3 = packed.reshape(B, N, 16)
    scores = pk3[:, :, 4]
    order = jnp.argsort(-scores, axis=1)
    srt = jnp.take_along_axis(pk3[:, :, 0:5], order[:, :, None], axis=1)
    pad = jnp.concatenate(
        [jnp.zeros((B, _NP - N, 4), jnp.float32),
         jnp.full((B, _NP - N, 1), -1.0, jnp.float32)], axis=2)
    srt = jnp.concatenate([srt, pad], axis=1)
    carr = jnp.moveaxis(srt, 2, 1)
    c1 = carr.reshape(B, 5, _NBLK, 1, _T)
    c2 = carr.reshape(B, 5, _NGRP, 1, _G)

    keep_f = pl.pallas_call(
        _nms_body,
        grid=(B,),
        in_specs=[
            pl.BlockSpec((None, 5, _NBLK, 1, _T), lambda b: (b, 0, 0, 0, 0)),
            pl.BlockSpec((None, 5, _NGRP, 1, _G), lambda b: (b, 0, 0, 0, 0)),
        ],
        out_specs=pl.BlockSpec((None, _NBLK, 1, _T), lambda b: (b, 0, 0, 0)),
        out_shape=jax.ShapeDtypeStruct((B, _NBLK, 1, _T), jnp.float32),
        scratch_shapes=[pltpu.VMEM((_NBLK, 1, _T), jnp.float32),
                        pltpu.VMEM((_NGRP, 1, _G), jnp.float32)],
    )(c1, c2)

    keep = keep_f.reshape(B, _NP)[:, :N] > 0.5
    pos = jnp.arange(N, dtype=jnp.int32)
    ck = jnp.cumsum(keep.astype(jnp.int32), axis=1)
    ktot = ck[:, -1:]
    rank = jnp.where(keep, ck - 1, ktot + pos[None, :] - ck)
    bidx = jnp.arange(B, dtype=jnp.int32)[:, None]
    sel = jnp.zeros((B, _MAXDET), jnp.int32).at[
        bidx, rank].set(order.astype(jnp.int32), mode="drop")
    vals = jnp.arange(_MAXDET, dtype=jnp.int32)[None, :] < ktot
    flat = sel + (jnp.arange(B, dtype=jnp.int32) * N)[:, None]
    flat_pad = jnp.concatenate(
        [flat, jnp.zeros((B, _NSEL - _MAXDET), jnp.int32)], axis=1).reshape(-1)
    table = jnp.concatenate(
        [logits.reshape(B * N, _C), packed,
         jnp.zeros((B * N, 128 - _C - 16), jnp.float32)], axis=1)
    rows = _sc_gather(flat_pad, table)
    rows = rows.reshape(B, _NSEL, 128)
    logs = rows[:, :_MAXDET, :_C]
    dets = rows[:, :_MAXDET, _C + 5:_C + 11]
    return (dets, logs, vals)
